# baseline (speedup 1.0000x reference)
import functools

import jax
import jax.numpy as jnp
from jax import lax
from jax.experimental import pallas as pl
from jax.experimental.pallas import tpu as pltpu

_TA = (((0,), (0,)), ((), ()))


def _fused_kernel(x_ref, wcat_ref, b3r_ref, b5r_ref, b1_ref,
                  w3_ref, b3_ref, w5_ref, b5_ref, bmp_ref, o_ref,
                  xpad, m1pad, ypad, p3, p5,
                  *, H, W, cin, cr3, cr5, c1, c3, c5, cpool, c0, L):
    HW = H * W
    f32 = jnp.float32
    bf16 = jnp.bfloat16
    NEG = jnp.array(-jnp.inf, bf16)

    xpad[:, c0:c0 + L] = x_ref[...].astype(bf16)

    lane = lax.broadcasted_iota(jnp.int32, (1, L), 1)
    pos = lane % HW
    xc = pos % W
    yc = pos // W
    okx = {dx: (xc + dx >= 0) & (xc + dx < W) for dx in (-2, -1, 1, 2)}
    oky = {dy: (yc + dy >= 0) & (yc + dy < H) for dy in (-2, -1, 1, 2)}

    def tap_mask(dy, dx):
        if dy == 0 and dx == 0:
            return None
        if dy == 0:
            return okx[dx]
        if dx == 0:
            return oky[dy]
        return okx[dx] & oky[dy]

    rr = cr3 + cr5
    wsh = wcat_ref[:, 0:rr + c1].astype(bf16)
    bsh = jnp.concatenate(
        [b3r_ref[...], b5r_ref[...], b1_ref[...]], axis=1).T
    xc_b = xpad[:, c0:c0 + L]
    y = lax.dot_general(wsh, xc_b, _TA, preferred_element_type=f32)
    y = jnp.maximum(y + bsh, 0.0)

    o_ref[0:c1, :] = y[rr:rr + c1]

    ypad[:, c0:c0 + L] = y[0:rr]
    for dy in range(-1, 2):
        for dx in range(-1, 2):
            t = (dy + 1) * 3 + (dx + 1)
            s = dy * W + dx
            win = ypad[0:cr3, c0 + s:c0 + s + L]
            m = tap_mask(dy, dx)
            if m is not None:
                win = jnp.where(m, win, 0.0)
            p3[t * cr3:(t + 1) * cr3, :] = win
    for dy in range(-2, 3):
        for dx in range(-2, 3):
            t = (dy + 2) * 5 + (dx + 2)
            s = dy * W + dx
            win = ypad[cr3:rr, c0 + s:c0 + s + L]
            m = tap_mask(dy, dx)
            if m is not None:
                win = jnp.where(m, win, 0.0)
            p5[t * cr5:(t + 1) * cr5, :] = win

    w3f = jnp.concatenate(
        [w3_ref[dy, dx] for dy in range(3) for dx in range(3)], axis=0)
    o3 = lax.dot_general(w3f, p3[...], _TA, preferred_element_type=f32)
    o3 = jnp.maximum(o3 + b3_ref[...].T, 0.0)
    o_ref[c1:c1 + c3, :] = o3
    w5f = jnp.concatenate(
        [w5_ref[dy, dx] for dy in range(5) for dx in range(5)], axis=0)
    o5 = lax.dot_general(w5f, p5[...], _TA, preferred_element_type=f32)
    o5 = jnp.maximum(o5 + b5_ref[...].T, 0.0)
    o_ref[c1 + c3:c1 + c3 + c5, :] = o5

    wl = jnp.where(okx[-1], xpad[:, c0 - 1:c0 - 1 + L], NEG)
    wr = jnp.where(okx[1], xpad[:, c0 + 1:c0 + 1 + L], NEG)
    m1 = jnp.maximum(jnp.maximum(wl, wr), xc_b)
    m1pad[:, c0:c0 + L] = m1
    vu = jnp.where(oky[-1], m1pad[:, c0 - W:c0 - W + L], NEG)
    vd = jnp.where(oky[1], m1pad[:, c0 + W:c0 + W + L], NEG)
    pooled = jnp.maximum(jnp.maximum(vu, vd), m1)

    omp = lax.dot_general(wcat_ref[:, rr + c1:].astype(bf16), pooled, _TA,
                          preferred_element_type=f32)
    omp = jnp.maximum(omp + bmp_ref[...].T, 0.0)
    cc = c1 + c3 + c5
    o_ref[cc:cc + cpool, :] = omp


def _inception_fused(x, w1, b1, w3r, b3r, w3, b3, w5r, b5r, w5, b5, wmp, bmp):
    N, cin, H, W = x.shape
    HW = H * W
    f32 = jnp.float32

    c1 = w1.shape[1]
    cr3 = w3r.shape[1]
    cr5 = w5r.shape[1]
    c3 = w3.shape[-1]
    c5 = w5.shape[-1]
    cpool = wmp.shape[1]
    ctot = c1 + c3 + c5 + cpool
    rr = cr3 + cr5

    P = next(p for p in (16, 8, 4, 2, 1) if N % p == 0)
    L = P * HW
    c0 = 128

    xt = jnp.transpose(x.reshape(N, cin, HW), (1, 0, 2)).reshape(cin, N * HW)
    wcat = jnp.concatenate([w3r, w5r, w1, wmp], axis=1)

    kern = functools.partial(
        _fused_kernel, H=H, W=W, cin=cin, cr3=cr3, cr5=cr5,
        c1=c1, c3=c3, c5=c5, cpool=cpool, c0=c0, L=L)

    hwp = L + 2 * c0
    full = lambda g: (0, 0)
    out = pl.pallas_call(
        kern,
        out_shape=jax.ShapeDtypeStruct((ctot, N * HW), f32),
        grid=(N // P,),
        in_specs=[
            pl.BlockSpec((cin, L), lambda g: (0, g)),
            pl.BlockSpec((cin, rr + c1 + cpool), full),
            pl.BlockSpec((1, cr3), full),
            pl.BlockSpec((1, cr5), full),
            pl.BlockSpec((1, c1), full),
            pl.BlockSpec((3, 3, cr3, c3), lambda g: (0, 0, 0, 0)),
            pl.BlockSpec((1, c3), full),
            pl.BlockSpec((5, 5, cr5, c5), lambda g: (0, 0, 0, 0)),
            pl.BlockSpec((1, c5), full),
            pl.BlockSpec((1, cpool), full),
        ],
        out_specs=pl.BlockSpec((ctot, L), lambda g: (0, g)),
        scratch_shapes=[
            pltpu.VMEM((cin, hwp), jnp.bfloat16),
            pltpu.VMEM((cin, hwp), jnp.bfloat16),
            pltpu.VMEM((rr, hwp), f32),
            pltpu.VMEM((9 * cr3, L), f32),
            pltpu.VMEM((25 * cr5, L), f32),
        ],
        compiler_params=pltpu.CompilerParams(
            dimension_semantics=("parallel",),
            vmem_limit_bytes=64 << 20),
    )(xt, wcat, b3r, b5r, b1, w3, b3, w5, b5, bmp)

    return jnp.transpose(out.reshape(ctot, N, H, W), (1, 0, 2, 3))


kernel = jax.jit(_inception_fused)


# speedup vs baseline: 1.0407x; 1.0407x over previous
import functools

import jax
import jax.numpy as jnp
from jax import lax
from jax.experimental import pallas as pl
from jax.experimental.pallas import tpu as pltpu

_TA = (((0,), (0,)), ((), ()))


def _fused_kernel(x_ref, wcat_ref, b3r_ref, b5r_ref, b1_ref,
                  w3_ref, b3_ref, w5_ref, b5_ref, bmp_ref, o_ref,
                  xpad, m1pad, ypad, p3, p5,
                  *, H, W, cin, cr3, cr5, c1, c3, c5, cpool, c0, L):
    HW = H * W
    f32 = jnp.float32
    bf16 = jnp.bfloat16
    NEG = jnp.array(-jnp.inf, bf16)

    xpad[:, c0:c0 + L] = x_ref[...].astype(bf16)

    lane = lax.broadcasted_iota(jnp.int32, (1, L), 1)
    pos = lane % HW
    xc = pos % W
    yc = pos // W
    okx = {dx: (xc + dx >= 0) & (xc + dx < W) for dx in (-2, -1, 1, 2)}
    oky = {dy: (yc + dy >= 0) & (yc + dy < H) for dy in (-2, -1, 1, 2)}

    def tap_mask(dy, dx):
        if dy == 0 and dx == 0:
            return None
        if dy == 0:
            return okx[dx]
        if dx == 0:
            return oky[dy]
        return okx[dx] & oky[dy]

    rr = cr3 + cr5
    wsh = wcat_ref[:, 0:rr + c1].astype(bf16)
    bsh = jnp.concatenate(
        [b3r_ref[...], b5r_ref[...], b1_ref[...]], axis=1).T
    xc_b = xpad[:, c0:c0 + L]
    y = lax.dot_general(wsh, xc_b, _TA, preferred_element_type=f32)
    y = jnp.maximum(y + bsh, 0.0)

    o_ref[0:c1, :] = y[rr:rr + c1]

    ypad[:, c0:c0 + L] = y[0:rr]
    for dy in range(-1, 2):
        for dx in range(-1, 2):
            t = (dy + 1) * 3 + (dx + 1)
            s = dy * W + dx
            win = ypad[0:cr3, c0 + s:c0 + s + L]
            m = tap_mask(dy, dx)
            if m is not None:
                win = jnp.where(m, win, 0.0)
            p3[t * cr3:(t + 1) * cr3, :] = win
    for dy in range(-2, 3):
        for dx in range(-2, 3):
            t = (dy + 2) * 5 + (dx + 2)
            s = dy * W + dx
            win = ypad[cr3:rr, c0 + s:c0 + s + L]
            m = tap_mask(dy, dx)
            if m is not None:
                win = jnp.where(m, win, 0.0)
            p5[t * cr5:(t + 1) * cr5, :] = win

    w3f = jnp.concatenate(
        [w3_ref[dy, dx] for dy in range(3) for dx in range(3)], axis=0)
    o3 = lax.dot_general(w3f, p3[...], _TA, preferred_element_type=f32)
    o3 = jnp.maximum(o3 + b3_ref[...].T, 0.0)
    o_ref[c1:c1 + c3, :] = o3
    w5f = jnp.concatenate(
        [w5_ref[dy, dx] for dy in range(5) for dx in range(5)], axis=0)
    o5 = lax.dot_general(w5f, p5[...], _TA, preferred_element_type=f32)
    o5 = jnp.maximum(o5 + b5_ref[...].T, 0.0)
    o_ref[c1 + c3:c1 + c3 + c5, :] = o5

    wl = jnp.where(okx[-1], xpad[:, c0 - 1:c0 - 1 + L], NEG)
    wr = jnp.where(okx[1], xpad[:, c0 + 1:c0 + 1 + L], NEG)
    m1 = jnp.maximum(jnp.maximum(wl, wr), xc_b)
    m1pad[:, c0:c0 + L] = m1
    vu = jnp.where(oky[-1], m1pad[:, c0 - W:c0 - W + L], NEG)
    vd = jnp.where(oky[1], m1pad[:, c0 + W:c0 + W + L], NEG)
    pooled = jnp.maximum(jnp.maximum(vu, vd), m1)

    omp = lax.dot_general(wcat_ref[:, rr + c1:].astype(bf16), pooled, _TA,
                          preferred_element_type=f32)
    omp = jnp.maximum(omp + bmp_ref[...].T, 0.0)
    cc = c1 + c3 + c5
    o_ref[cc:cc + cpool, :] = omp


def _inception_fused(x, w1, b1, w3r, b3r, w3, b3, w5r, b5r, w5, b5, wmp, bmp):
    N, cin, H, W = x.shape
    HW = H * W
    f32 = jnp.float32

    c1 = w1.shape[1]
    cr3 = w3r.shape[1]
    cr5 = w5r.shape[1]
    c3 = w3.shape[-1]
    c5 = w5.shape[-1]
    cpool = wmp.shape[1]
    ctot = c1 + c3 + c5 + cpool
    rr = cr3 + cr5

    P = next(p for p in (32, 16, 8, 4, 2, 1) if N % p == 0)
    L = P * HW
    c0 = 128

    xt = jnp.transpose(x.reshape(N, cin, HW), (1, 0, 2)).reshape(cin, N * HW)
    wcat = jnp.concatenate([w3r, w5r, w1, wmp], axis=1)

    kern = functools.partial(
        _fused_kernel, H=H, W=W, cin=cin, cr3=cr3, cr5=cr5,
        c1=c1, c3=c3, c5=c5, cpool=cpool, c0=c0, L=L)

    hwp = L + 2 * c0
    full = lambda g: (0, 0)
    out = pl.pallas_call(
        kern,
        out_shape=jax.ShapeDtypeStruct((ctot, N * HW), f32),
        grid=(N // P,),
        in_specs=[
            pl.BlockSpec((cin, L), lambda g: (0, g)),
            pl.BlockSpec((cin, rr + c1 + cpool), full),
            pl.BlockSpec((1, cr3), full),
            pl.BlockSpec((1, cr5), full),
            pl.BlockSpec((1, c1), full),
            pl.BlockSpec((3, 3, cr3, c3), lambda g: (0, 0, 0, 0)),
            pl.BlockSpec((1, c3), full),
            pl.BlockSpec((5, 5, cr5, c5), lambda g: (0, 0, 0, 0)),
            pl.BlockSpec((1, c5), full),
            pl.BlockSpec((1, cpool), full),
        ],
        out_specs=pl.BlockSpec((ctot, L), lambda g: (0, g)),
        scratch_shapes=[
            pltpu.VMEM((cin, hwp), jnp.bfloat16),
            pltpu.VMEM((cin, hwp), jnp.bfloat16),
            pltpu.VMEM((rr, hwp), f32),
            pltpu.VMEM((9 * cr3, L), f32),
            pltpu.VMEM((25 * cr5, L), f32),
        ],
        compiler_params=pltpu.CompilerParams(
            dimension_semantics=("parallel",),
            vmem_limit_bytes=64 << 20),
    )(xt, wcat, b3r, b5r, b1, w3, b3, w5, b5, bmp)

    return jnp.transpose(out.reshape(ctot, N, H, W), (1, 0, 2, 3))


kernel = jax.jit(_inception_fused)
